# Initial kernel scaffold; baseline (speedup 1.0000x reference)
import jax
import jax.numpy as jnp
from jax.experimental import pallas as pl


def kernel(frac_coords, lattice_matrices, batch_id):
    raise NotImplementedError("write your pallas kernel here")


# speedup vs baseline: 3.9533x; 3.9533x over previous
import functools

import jax
import jax.numpy as jnp
from jax import lax
from jax.experimental import pallas as pl
from jax.experimental.pallas import tpu as pltpu
from jax.experimental.pallas import tpu_sc as plsc

N_NODES = 100000
B_SAMPLES = 64

_NUM_WORKERS = 32
_LANES = 16
_CHUNK = 3136
_N_PAD = _NUM_WORKERS * _CHUNK
_VECS = _CHUNK // _LANES


def _sc_body(lat_hbm, frac_hbm, bid_hbm, out_hbm,
             table_v, bid_v, frac_v, out_v):
    wid = lax.axis_index("s") * 2 + lax.axis_index("c")
    base = wid * _CHUNK

    pltpu.sync_copy(lat_hbm, table_v)
    pltpu.sync_copy(bid_hbm.at[pl.ds(base, _CHUNK)], bid_v)
    pltpu.sync_copy(frac_hbm.at[pl.ds(base * 3, _CHUNK * 3)], frac_v)

    lane = lax.iota(jnp.int32, _LANES)

    def step(i, carry):
        ids = lane + i * _LANES
        ids3 = ids * 3
        b9 = bid_v[pl.ds(i * _LANES, _LANES)] * 9
        f0 = plsc.load_gather(frac_v, [ids3])
        f1 = plsc.load_gather(frac_v, [ids3 + 1])
        f2 = plsc.load_gather(frac_v, [ids3 + 2])
        for k in range(3):
            a0 = plsc.load_gather(table_v, [b9 + k])
            a1 = plsc.load_gather(table_v, [b9 + (3 + k)])
            a2 = plsc.load_gather(table_v, [b9 + (6 + k)])
            plsc.store_scatter(out_v, [ids3 + k], f0 * a0 + f1 * a1 + f2 * a2)
        return carry

    lax.fori_loop(0, _VECS, step, 0, unroll=4)

    pltpu.sync_copy(out_v, out_hbm.at[pl.ds(base * 3, _CHUNK * 3)])


@functools.partial(jax.jit, static_argnames=())
def _run(lat_flat, frac_flat, bid_pad):
    mesh = plsc.VectorSubcoreMesh(core_axis_name="c", subcore_axis_name="s")
    return pl.kernel(
        _sc_body,
        out_type=jax.ShapeDtypeStruct((_N_PAD * 3,), jnp.float32),
        mesh=mesh,
        scratch_types=[
            pltpu.VMEM((B_SAMPLES * 9,), jnp.float32),
            pltpu.VMEM((_CHUNK,), jnp.int32),
            pltpu.VMEM((_CHUNK * 3,), jnp.float32),
            pltpu.VMEM((_CHUNK * 3,), jnp.float32),
        ],
        compiler_params=pltpu.CompilerParams(needs_layout_passes=False),
    )(lat_flat, frac_flat, bid_pad)


def kernel(frac_coords, lattice_matrices, batch_id):
    n = frac_coords.shape[0]
    pad = _N_PAD - n
    frac_flat = jnp.pad(frac_coords, ((0, pad), (0, 0))).reshape(-1)
    bid_pad = jnp.pad(batch_id.astype(jnp.int32), (0, pad))
    lat_flat = lattice_matrices.reshape(-1).astype(jnp.float32)
    out_flat = _run(lat_flat, frac_flat, bid_pad)
    return out_flat.reshape(_N_PAD, 3)[:n]


# speedup vs baseline: 6.9561x; 1.7596x over previous
import jax
import jax.numpy as jnp
from jax import lax
from jax.experimental import pallas as pl
from jax.experimental.pallas import tpu as pltpu
from jax.experimental.pallas import tpu_sc as plsc

N_NODES = 100000
B_SAMPLES = 64

_LANES = 16
_TILES_TOTAL = N_NODES // 8
_WTILES = 392
_BLK = 56
_NBLK = _WTILES // _BLK
_BLK_NODES = _BLK * 8
_BLK_VECS = _BLK_NODES // _LANES
_CHUNK = _WTILES * 8


def _sc_body(frac_hbm, lat_hbm, bid_hbm, out_hbm,
             table_v, bid_v, f_v, o_v):
    wid = lax.axis_index("s") * 2 + lax.axis_index("c")
    tbase = lax.min(wid * _WTILES, _TILES_TOTAL - _WTILES)

    frac_t = frac_hbm.reshape(_TILES_TOTAL, 8, 3)
    out_t = out_hbm.reshape(_TILES_TOTAL, 8, 3)

    pltpu.sync_copy(lat_hbm, table_v)
    pltpu.sync_copy(bid_hbm.at[pl.ds(tbase * 8, _CHUNK)], bid_v)

    lane = lax.iota(jnp.int32, _LANES)
    sub = lane & 7
    tof = lane >> 3

    def make_step(blk):
        def step(i, carry):
            t = tof + i * 2
            b9 = bid_v[pl.ds(blk * _BLK_NODES + i * _LANES, _LANES)] * 9
            f0 = plsc.load_gather(f_v, [t, sub, sub * 0])
            f1 = plsc.load_gather(f_v, [t, sub, sub * 0 + 1])
            f2 = plsc.load_gather(f_v, [t, sub, sub * 0 + 2])
            for k in range(3):
                a0 = plsc.load_gather(table_v, [b9 + k])
                a1 = plsc.load_gather(table_v, [b9 + (3 + k)])
                a2 = plsc.load_gather(table_v, [b9 + (6 + k)])
                plsc.store_scatter(o_v, [t, sub, sub * 0 + k],
                                   f0 * a0 + f1 * a1 + f2 * a2)
            return carry
        return step

    for blk in range(_NBLK):
        gt0 = tbase + blk * _BLK
        pltpu.sync_copy(frac_t.at[pl.ds(gt0, _BLK)], f_v)
        lax.fori_loop(0, _BLK_VECS, make_step(blk), 0, unroll=4)
        pltpu.sync_copy(o_v, out_t.at[pl.ds(gt0, _BLK)])


@jax.jit
def _run(frac_coords, lat_flat, batch_id):
    mesh = plsc.VectorSubcoreMesh(core_axis_name="c", subcore_axis_name="s")
    return pl.kernel(
        _sc_body,
        out_type=jax.ShapeDtypeStruct((N_NODES, 3), jnp.float32),
        mesh=mesh,
        scratch_types=[
            pltpu.VMEM((B_SAMPLES * 9,), jnp.float32),
            pltpu.VMEM((_CHUNK,), jnp.int32),
            pltpu.VMEM((_BLK, 8, 3), jnp.float32),
            pltpu.VMEM((_BLK, 8, 3), jnp.float32),
        ],
        compiler_params=pltpu.CompilerParams(needs_layout_passes=False),
    )(frac_coords, lat_flat, batch_id)


def kernel(frac_coords, lattice_matrices, batch_id):
    return _run(frac_coords, lattice_matrices.reshape(-1).astype(jnp.float32),
                batch_id.astype(jnp.int32))


# speedup vs baseline: 21.3871x; 3.0746x over previous
import jax
import jax.numpy as jnp
from jax import lax
from jax.experimental import pallas as pl
from jax.experimental.pallas import tpu as pltpu
from jax.experimental.pallas import tpu_sc as plsc

N_NODES = 100000
B_SAMPLES = 64

_LANES = 16
_CHUNK = 3136
_VECS = _CHUNK // _LANES


def _sc_body(fx, fy, fz, lat_hbm, bid_hbm, ox, oy, oz,
             table_v, bid_v, fx_v, fy_v, fz_v, ox_v, oy_v, oz_v):
    wid = lax.axis_index("s") * 2 + lax.axis_index("c")
    base = lax.min(wid * _CHUNK, N_NODES - _CHUNK)

    pltpu.sync_copy(lat_hbm, table_v)
    pltpu.sync_copy(bid_hbm.at[pl.ds(base, _CHUNK)], bid_v)
    pltpu.sync_copy(fx.at[pl.ds(base, _CHUNK)], fx_v)
    pltpu.sync_copy(fy.at[pl.ds(base, _CHUNK)], fy_v)
    pltpu.sync_copy(fz.at[pl.ds(base, _CHUNK)], fz_v)

    def step(i, carry):
        sl = pl.ds(i * _LANES, _LANES)
        b9 = bid_v[sl] * 9
        f0 = fx_v[sl]
        f1 = fy_v[sl]
        f2 = fz_v[sl]
        o_refs = (ox_v, oy_v, oz_v)
        for k in range(3):
            a0 = plsc.load_gather(table_v, [b9 + k])
            a1 = plsc.load_gather(table_v, [b9 + (3 + k)])
            a2 = plsc.load_gather(table_v, [b9 + (6 + k)])
            o_refs[k][sl] = f0 * a0 + f1 * a1 + f2 * a2
        return carry

    lax.fori_loop(0, _VECS, step, 0, unroll=4)

    pltpu.sync_copy(ox_v, ox.at[pl.ds(base, _CHUNK)])
    pltpu.sync_copy(oy_v, oy.at[pl.ds(base, _CHUNK)])
    pltpu.sync_copy(oz_v, oz.at[pl.ds(base, _CHUNK)])


@jax.jit
def _run(frac_coords, lattice_matrices, batch_id):
    mesh = plsc.VectorSubcoreMesh(core_axis_name="c", subcore_axis_name="s")
    col = jax.ShapeDtypeStruct((N_NODES,), jnp.float32)
    ox, oy, oz = pl.kernel(
        _sc_body,
        out_type=(col, col, col),
        mesh=mesh,
        scratch_types=[
            pltpu.VMEM((B_SAMPLES * 9,), jnp.float32),
            pltpu.VMEM((_CHUNK,), jnp.int32),
        ] + [pltpu.VMEM((_CHUNK,), jnp.float32)] * 6,
        compiler_params=pltpu.CompilerParams(needs_layout_passes=False),
    )(frac_coords[:, 0], frac_coords[:, 1], frac_coords[:, 2],
      lattice_matrices.reshape(-1).astype(jnp.float32),
      batch_id.astype(jnp.int32))
    return jnp.stack([ox, oy, oz], axis=1)


def kernel(frac_coords, lattice_matrices, batch_id):
    return _run(frac_coords, lattice_matrices, batch_id)


# speedup vs baseline: 24.7082x; 1.1553x over previous
import jax
import jax.numpy as jnp
from jax import lax
from jax.experimental import pallas as pl
from jax.experimental.pallas import tpu as pltpu
from jax.experimental.pallas import tpu_sc as plsc

N_NODES = 100000
B_SAMPLES = 64

_LANES = 16
_CHUNK = 3136
_VECS = _CHUNK // _LANES


def _sc_body(ft, lat_hbm, bid_hbm, ot,
             table_v, bid_v, fx_v, fy_v, fz_v, ox_v, oy_v, oz_v):
    wid = lax.axis_index("s") * 2 + lax.axis_index("c")
    base = lax.min(wid * _CHUNK, N_NODES - _CHUNK)

    pltpu.sync_copy(lat_hbm, table_v)
    pltpu.sync_copy(bid_hbm.at[pl.ds(base, _CHUNK)], bid_v)
    pltpu.sync_copy(ft.at[pl.ds(base, _CHUNK)], fx_v)
    pltpu.sync_copy(ft.at[pl.ds(N_NODES + base, _CHUNK)], fy_v)
    pltpu.sync_copy(ft.at[pl.ds(2 * N_NODES + base, _CHUNK)], fz_v)

    def step(i, carry):
        sl = pl.ds(i * _LANES, _LANES)
        b9 = bid_v[sl] * 9
        f0 = fx_v[sl]
        f1 = fy_v[sl]
        f2 = fz_v[sl]
        o_refs = (ox_v, oy_v, oz_v)
        for k in range(3):
            a0 = plsc.load_gather(table_v, [b9 + k])
            a1 = plsc.load_gather(table_v, [b9 + (3 + k)])
            a2 = plsc.load_gather(table_v, [b9 + (6 + k)])
            o_refs[k][sl] = f0 * a0 + f1 * a1 + f2 * a2
        return carry

    lax.fori_loop(0, _VECS, step, 0, unroll=4)

    pltpu.sync_copy(ox_v, ot.at[pl.ds(base, _CHUNK)])
    pltpu.sync_copy(oy_v, ot.at[pl.ds(N_NODES + base, _CHUNK)])
    pltpu.sync_copy(oz_v, ot.at[pl.ds(2 * N_NODES + base, _CHUNK)])


@jax.jit
def _run(frac_coords, lattice_matrices, batch_id):
    mesh = plsc.VectorSubcoreMesh(core_axis_name="c", subcore_axis_name="s")
    ot = pl.kernel(
        _sc_body,
        out_type=jax.ShapeDtypeStruct((3 * N_NODES,), jnp.float32),
        mesh=mesh,
        scratch_types=[
            pltpu.VMEM((B_SAMPLES * 9,), jnp.float32),
            pltpu.VMEM((_CHUNK,), jnp.int32),
        ] + [pltpu.VMEM((_CHUNK,), jnp.float32)] * 6,
        compiler_params=pltpu.CompilerParams(needs_layout_passes=False),
    )(frac_coords.T.reshape(-1),
      lattice_matrices.reshape(-1).astype(jnp.float32),
      batch_id.astype(jnp.int32))
    return ot.reshape(3, N_NODES).T


def kernel(frac_coords, lattice_matrices, batch_id):
    return _run(frac_coords, lattice_matrices, batch_id)


# speedup vs baseline: 27.9276x; 1.1303x over previous
import jax
import jax.numpy as jnp
from jax import lax
from jax.experimental import pallas as pl
from jax.experimental.pallas import tpu as pltpu
from jax.experimental.pallas import tpu_sc as plsc

N_NODES = 100000
B_SAMPLES = 64

_LANES = 16
_CHUNK = 3136
_VECS = _CHUNK // _LANES


def _sc_body(ft, lat_hbm, bid_hbm, ot,
             table_v, bid_v, fx_v, fy_v, fz_v, ox_v, oy_v, oz_v):
    wid = lax.axis_index("s") * 2 + lax.axis_index("c")
    base = lax.min(wid * _CHUNK, N_NODES - _CHUNK)

    pltpu.sync_copy(lat_hbm, table_v)
    pltpu.sync_copy(bid_hbm.at[pl.ds(base, _CHUNK)], bid_v)
    pltpu.sync_copy(ft.at[pl.ds(base, _CHUNK)], fx_v)
    pltpu.sync_copy(ft.at[pl.ds(N_NODES + base, _CHUNK)], fy_v)
    pltpu.sync_copy(ft.at[pl.ds(2 * N_NODES + base, _CHUNK)], fz_v)

    @plsc.parallel_loop(0, _VECS, unroll=4)
    def step(i):
        sl = pl.ds(i * _LANES, _LANES)
        b9 = bid_v[sl] * 9
        f0 = fx_v[sl]
        f1 = fy_v[sl]
        f2 = fz_v[sl]
        o_refs = (ox_v, oy_v, oz_v)
        for k in range(3):
            a0 = plsc.load_gather(table_v, [b9 + k])
            a1 = plsc.load_gather(table_v, [b9 + (3 + k)])
            a2 = plsc.load_gather(table_v, [b9 + (6 + k)])
            o_refs[k][sl] = f0 * a0 + f1 * a1 + f2 * a2

    pltpu.sync_copy(ox_v, ot.at[pl.ds(base, _CHUNK)])
    pltpu.sync_copy(oy_v, ot.at[pl.ds(N_NODES + base, _CHUNK)])
    pltpu.sync_copy(oz_v, ot.at[pl.ds(2 * N_NODES + base, _CHUNK)])


@jax.jit
def _run(frac_coords, lattice_matrices, batch_id):
    mesh = plsc.VectorSubcoreMesh(core_axis_name="c", subcore_axis_name="s")
    ot = pl.kernel(
        _sc_body,
        out_type=jax.ShapeDtypeStruct((3 * N_NODES,), jnp.float32),
        mesh=mesh,
        scratch_types=[
            pltpu.VMEM((B_SAMPLES * 9,), jnp.float32),
            pltpu.VMEM((_CHUNK,), jnp.int32),
        ] + [pltpu.VMEM((_CHUNK,), jnp.float32)] * 6,
        compiler_params=pltpu.CompilerParams(needs_layout_passes=False),
    )(frac_coords.T.reshape(-1),
      lattice_matrices.reshape(-1).astype(jnp.float32),
      batch_id.astype(jnp.int32))
    return ot.reshape(3, N_NODES).T


def kernel(frac_coords, lattice_matrices, batch_id):
    return _run(frac_coords, lattice_matrices, batch_id)
